# initial kernel scaffold (unmeasured)
import jax
import jax.numpy as jnp
from jax import lax
from jax.experimental import pallas as pl
from jax.experimental.pallas import tpu as pltpu

N_DEV = 4
H = 32
HL = H // N_DEV
DH = 128
SQ = 256
SKV_LOC = 4096
DM = 1024
SCALE = 0.08838834764831843
QB = 64
NQB = SQ // QB


def _mesh_peers(my):
    return [(my + d) % N_DEV for d in range(1, N_DEV)]


def _entry_barrier(my):
    barrier = pltpu.get_barrier_semaphore()
    for p in _mesh_peers(my):
        pl.semaphore_signal(
            barrier, inc=1, device_id=(p,),
            device_id_type=pl.DeviceIdType.MESH,
        )
    pl.semaphore_wait(barrier, N_DEV - 1)


def _qproj_allgather(x, Wq):

    def body(x_ref, wq_ref, q_ref, send_sems, recv_sems):
        my = lax.axis_index("i")
        _entry_barrier(my)

        xq = x_ref[0].astype(jnp.bfloat16)
        wq = wq_ref[...].astype(jnp.bfloat16)
        q = lax.dot(xq, wq, preferred_element_type=jnp.float32)
        q_ref[my] = (q * SCALE).astype(jnp.bfloat16)

        sends = []
        for d, p in enumerate(_mesh_peers(my)):
            rdma = pltpu.make_async_remote_copy(
                src_ref=q_ref.at[my],
                dst_ref=q_ref.at[my],
                send_sem=send_sems.at[d],
                recv_sem=recv_sems.at[my],
                device_id=(p,),
                device_id_type=pl.DeviceIdType.MESH,
            )
            rdma.start()
            sends.append(rdma)
        for p in _mesh_peers(my):
            recv = pltpu.make_async_remote_copy(
                src_ref=q_ref.at[p],
                dst_ref=q_ref.at[p],
                send_sem=send_sems.at[0],
                recv_sem=recv_sems.at[p],
                device_id=(p,),
                device_id_type=pl.DeviceIdType.MESH,
            )
            recv.wait_recv()
        for rdma in sends:
            rdma.wait_send()

    return pl.pallas_call(
        body,
        out_shape=jax.ShapeDtypeStruct((N_DEV, SQ, DM), jnp.bfloat16),
        in_specs=[
            pl.BlockSpec(memory_space=pltpu.VMEM),
            pl.BlockSpec(memory_space=pltpu.VMEM),
        ],
        out_specs=pl.BlockSpec(memory_space=pltpu.VMEM),
        scratch_shapes=[
            pltpu.SemaphoreType.DMA((N_DEV - 1,)),
            pltpu.SemaphoreType.DMA((N_DEV,)),
        ],
        compiler_params=pltpu.CompilerParams(collective_id=0),
    )(x, Wq)


def _local_attn(q_all, K_ext, V_ext):

    def body(q_ref, k_ref, v_ref, num_ref, den_ref):
        q = q_ref[0]
        k = k_ref[0, :, 0, :].astype(jnp.bfloat16)
        v = v_ref[0, :, 0, :].astype(jnp.bfloat16)
        k4 = k.reshape(SKV_LOC // (4 * QB), 4, QB, DH)
        v4 = v.reshape(SKV_LOC // (4 * QB), 4, QB, DH)
        for qb in range(NQB):
            qblk = q[qb * QB:(qb + 1) * QB, :]
            ksel = k4[:, qb].reshape(-1, DH)
            vsel = v4[:, qb].reshape(-1, DH)
            s = lax.dot_general(
                qblk, ksel, (((1,), (1,)), ((), ())),
                preferred_element_type=jnp.float32,
            )
            w = jnp.exp(s)
            den_ref[0, qb * QB:(qb + 1) * QB] = jnp.sum(w, axis=1)
            num_ref[0, qb * QB:(qb + 1) * QB, :] = lax.dot(
                w.astype(jnp.bfloat16), vsel,
                preferred_element_type=jnp.float32,
            )

    return pl.pallas_call(
        body,
        grid=(H,),
        in_specs=[
            pl.BlockSpec((1, SQ, DH), lambda h: (h // HL, 0, h % HL)),
            pl.BlockSpec((1, SKV_LOC, 1, DH), lambda h: (0, 0, h, 0)),
            pl.BlockSpec((1, SKV_LOC, 1, DH), lambda h: (0, 0, h, 0)),
        ],
        out_specs=[
            pl.BlockSpec((1, SQ, DH), lambda h: (h, 0, 0)),
            pl.BlockSpec((1, SQ), lambda h: (h, 0)),
        ],
        out_shape=[
            jax.ShapeDtypeStruct((H, SQ, DH), jnp.float32),
            jax.ShapeDtypeStruct((H, SQ), jnp.float32),
        ],
    )(q_all, K_ext, V_ext)


def _combine(num, den, Wo):

    def body(num_ref, den_ref, wo_ref, out_ref,
             numr, denr, outr,
             nsend, nrecv, dsend, drecv, osend, orecv):
        my = lax.axis_index("i")
        _entry_barrier(my)

        sends = []
        for d, p in enumerate(_mesh_peers(my)):
            rn = pltpu.make_async_remote_copy(
                src_ref=num_ref.at[pl.ds(p * HL, HL)],
                dst_ref=numr.at[my],
                send_sem=nsend.at[d], recv_sem=nrecv.at[my],
                device_id=(p,), device_id_type=pl.DeviceIdType.MESH,
            )
            rn.start()
            rd = pltpu.make_async_remote_copy(
                src_ref=den_ref.at[pl.ds(p * HL, HL)],
                dst_ref=denr.at[my],
                send_sem=dsend.at[d], recv_sem=drecv.at[my],
                device_id=(p,), device_id_type=pl.DeviceIdType.MESH,
            )
            rd.start()
            sends += [rn, rd]

        acc_n = num_ref[pl.ds(my * HL, HL)]
        acc_d = den_ref[pl.ds(my * HL, HL)]
        for p in _mesh_peers(my):
            rn = pltpu.make_async_remote_copy(
                src_ref=num_ref.at[pl.ds(0, HL)], dst_ref=numr.at[p],
                send_sem=nsend.at[0], recv_sem=nrecv.at[p],
                device_id=(p,), device_id_type=pl.DeviceIdType.MESH,
            )
            rn.wait_recv()
            rd = pltpu.make_async_remote_copy(
                src_ref=den_ref.at[pl.ds(0, HL)], dst_ref=denr.at[p],
                send_sem=dsend.at[0], recv_sem=drecv.at[p],
                device_id=(p,), device_id_type=pl.DeviceIdType.MESH,
            )
            rd.wait_recv()
            acc_n = acc_n + numr[p]
            acc_d = acc_d + denr[p]

        ctx = (acc_n / acc_d[:, :, None]).astype(jnp.bfloat16)
        wo = wo_ref[...].astype(jnp.bfloat16)
        out_acc = jnp.zeros((SQ, DM), jnp.float32)
        for h in range(HL):
            out_acc = out_acc + lax.dot(
                ctx[h], wo[h * DH:(h + 1) * DH, :],
                preferred_element_type=jnp.float32,
            )
        outr[my] = out_acc

        osends = []
        for d, p in enumerate(_mesh_peers(my)):
            ro = pltpu.make_async_remote_copy(
                src_ref=outr.at[my], dst_ref=outr.at[my],
                send_sem=osend.at[d], recv_sem=orecv.at[my],
                device_id=(p,), device_id_type=pl.DeviceIdType.MESH,
            )
            ro.start()
            osends.append(ro)
        total = out_acc
        for p in _mesh_peers(my):
            ro = pltpu.make_async_remote_copy(
                src_ref=outr.at[p], dst_ref=outr.at[p],
                send_sem=osend.at[0], recv_sem=orecv.at[p],
                device_id=(p,), device_id_type=pl.DeviceIdType.MESH,
            )
            ro.wait_recv()
            total = total + outr[p]
        out_ref[0] = total

        for rdma in sends + osends:
            rdma.wait_send()

    return pl.pallas_call(
        body,
        out_shape=jax.ShapeDtypeStruct((1, SQ, DM), jnp.float32),
        in_specs=[
            pl.BlockSpec(memory_space=pltpu.VMEM),
            pl.BlockSpec(memory_space=pltpu.VMEM),
            pl.BlockSpec(memory_space=pltpu.VMEM),
        ],
        out_specs=pl.BlockSpec(memory_space=pltpu.VMEM),
        scratch_shapes=[
            pltpu.VMEM((N_DEV, HL, SQ, DH), jnp.float32),
            pltpu.VMEM((N_DEV, HL, SQ), jnp.float32),
            pltpu.VMEM((N_DEV, SQ, DM), jnp.float32),
            pltpu.SemaphoreType.DMA((N_DEV - 1,)),
            pltpu.SemaphoreType.DMA((N_DEV,)),
            pltpu.SemaphoreType.DMA((N_DEV - 1,)),
            pltpu.SemaphoreType.DMA((N_DEV,)),
            pltpu.SemaphoreType.DMA((N_DEV - 1,)),
            pltpu.SemaphoreType.DMA((N_DEV,)),
        ],
        compiler_params=pltpu.CompilerParams(collective_id=1),
    )(num, den, Wo)


def kernel(x, Wq, K_ext, V_ext, Wo):
    q_all = _qproj_allgather(x, Wq)
    num, den = _local_attn(q_all, K_ext, V_ext)
    return _combine(num, den, Wo)


# baseline (device time: 227537 ns/iter reference)
import jax
import jax.numpy as jnp
from jax import lax
from jax.experimental import pallas as pl
from jax.experimental.pallas import tpu as pltpu

N_DEV = 4
H = 32
HL = H // N_DEV
DH = 128
SQ = 256
SKV_LOC = 4096
DM = 1024
SCALE = 0.08838834764831843
QB = 64
NQB = SQ // QB


def _mesh_peers(my):
    return [(my + d) % N_DEV for d in range(1, N_DEV)]


def _entry_barrier(my):
    barrier = pltpu.get_barrier_semaphore()
    for p in _mesh_peers(my):
        pl.semaphore_signal(
            barrier, inc=1, device_id=(p,),
            device_id_type=pl.DeviceIdType.MESH,
        )
    pl.semaphore_wait(barrier, N_DEV - 1)


def _qproj_allgather(x, Wq):

    def body(x_ref, wq_ref, q_ref, send_sems, recv_sems):
        my = lax.axis_index("i")
        _entry_barrier(my)

        xq = x_ref[0].astype(jnp.bfloat16)
        wq = wq_ref[...].astype(jnp.bfloat16)
        q = lax.dot(xq, wq, preferred_element_type=jnp.float32)
        q_ref[my] = (q * SCALE).astype(jnp.bfloat16)

        sends = []
        for d, p in enumerate(_mesh_peers(my)):
            rdma = pltpu.make_async_remote_copy(
                src_ref=q_ref.at[my],
                dst_ref=q_ref.at[my],
                send_sem=send_sems.at[d],
                recv_sem=recv_sems.at[my],
                device_id=(p,),
                device_id_type=pl.DeviceIdType.MESH,
            )
            rdma.start()
            sends.append(rdma)
        for p in _mesh_peers(my):
            recv = pltpu.make_async_remote_copy(
                src_ref=q_ref.at[p],
                dst_ref=q_ref.at[p],
                send_sem=send_sems.at[0],
                recv_sem=recv_sems.at[p],
                device_id=(p,),
                device_id_type=pl.DeviceIdType.MESH,
            )
            recv.wait_recv()
        for rdma in sends:
            rdma.wait_send()

    return pl.pallas_call(
        body,
        out_shape=jax.ShapeDtypeStruct((N_DEV, SQ, DM), jnp.bfloat16),
        in_specs=[
            pl.BlockSpec(memory_space=pltpu.VMEM),
            pl.BlockSpec(memory_space=pltpu.VMEM),
        ],
        out_specs=pl.BlockSpec(memory_space=pltpu.VMEM),
        scratch_shapes=[
            pltpu.SemaphoreType.DMA((N_DEV - 1,)),
            pltpu.SemaphoreType.DMA((N_DEV,)),
        ],
        compiler_params=pltpu.CompilerParams(collective_id=0),
    )(x, Wq)


def _local_attn(q_all, K_ext, V_ext):

    def body(q_ref, k_ref, v_ref, num_ref, den_ref):
        q = q_ref[0]
        k = k_ref[...].astype(jnp.bfloat16)
        v = v_ref[...].astype(jnp.bfloat16)
        k4 = k.reshape(SKV_LOC // (4 * QB), 4, QB, DH)
        v4 = v.reshape(SKV_LOC // (4 * QB), 4, QB, DH)
        for qb in range(NQB):
            qblk = q[qb * QB:(qb + 1) * QB, :]
            ksel = k4[:, qb].reshape(-1, DH)
            vsel = v4[:, qb].reshape(-1, DH)
            s = lax.dot_general(
                qblk, ksel, (((1,), (1,)), ((), ())),
                preferred_element_type=jnp.float32,
            )
            w = jnp.exp(s)
            den_ref[0, 0, qb * QB:(qb + 1) * QB] = jnp.sum(w, axis=1)
            num_ref[0, qb * QB:(qb + 1) * QB, :] = lax.dot(
                w.astype(jnp.bfloat16), vsel,
                preferred_element_type=jnp.float32,
            )

    return pl.pallas_call(
        body,
        grid=(H,),
        in_specs=[
            pl.BlockSpec((1, SQ, DH), lambda h: (h // HL, 0, h % HL)),
            pl.BlockSpec((SKV_LOC, DH), lambda h: (0, h)),
            pl.BlockSpec((SKV_LOC, DH), lambda h: (0, h)),
        ],
        out_specs=[
            pl.BlockSpec((1, SQ, DH), lambda h: (h, 0, 0)),
            pl.BlockSpec((1, 8, SQ), lambda h: (h, 0, 0)),
        ],
        out_shape=[
            jax.ShapeDtypeStruct((H, SQ, DH), jnp.float32),
            jax.ShapeDtypeStruct((H, 8, SQ), jnp.float32),
        ],
    )(
        q_all,
        K_ext.reshape(SKV_LOC, H * DH),
        V_ext.reshape(SKV_LOC, H * DH),
    )


def _combine(num, den, Wo):

    def body(num_ref, den_ref, wo_ref, out_ref,
             numr, denr, outr,
             nsend, nrecv, dsend, drecv, osend, orecv):
        my = lax.axis_index("i")
        _entry_barrier(my)

        sends = []
        for d, p in enumerate(_mesh_peers(my)):
            rn = pltpu.make_async_remote_copy(
                src_ref=num_ref.at[pl.ds(p * HL, HL)],
                dst_ref=numr.at[my],
                send_sem=nsend.at[d], recv_sem=nrecv.at[my],
                device_id=(p,), device_id_type=pl.DeviceIdType.MESH,
            )
            rn.start()
            rd = pltpu.make_async_remote_copy(
                src_ref=den_ref.at[pl.ds(p * HL, HL)],
                dst_ref=denr.at[my],
                send_sem=dsend.at[d], recv_sem=drecv.at[my],
                device_id=(p,), device_id_type=pl.DeviceIdType.MESH,
            )
            rd.start()
            sends += [rn, rd]

        acc_n = num_ref[pl.ds(my * HL, HL)]
        acc_d = den_ref[pl.ds(my * HL, HL), 0, :]
        for p in _mesh_peers(my):
            rn = pltpu.make_async_remote_copy(
                src_ref=num_ref.at[pl.ds(0, HL)], dst_ref=numr.at[p],
                send_sem=nsend.at[0], recv_sem=nrecv.at[p],
                device_id=(p,), device_id_type=pl.DeviceIdType.MESH,
            )
            rn.wait_recv()
            rd = pltpu.make_async_remote_copy(
                src_ref=den_ref.at[pl.ds(0, HL)], dst_ref=denr.at[p],
                send_sem=dsend.at[0], recv_sem=drecv.at[p],
                device_id=(p,), device_id_type=pl.DeviceIdType.MESH,
            )
            rd.wait_recv()
            acc_n = acc_n + numr[p]
            acc_d = acc_d + denr[p, :, 0, :]

        ctx = (acc_n / acc_d[:, :, None]).astype(jnp.bfloat16)
        wo = wo_ref[...].astype(jnp.bfloat16)
        out_acc = jnp.zeros((SQ, DM), jnp.float32)
        for h in range(HL):
            out_acc = out_acc + lax.dot(
                ctx[h], wo[h * DH:(h + 1) * DH, :],
                preferred_element_type=jnp.float32,
            )
        outr[my] = out_acc

        osends = []
        for d, p in enumerate(_mesh_peers(my)):
            ro = pltpu.make_async_remote_copy(
                src_ref=outr.at[my], dst_ref=outr.at[my],
                send_sem=osend.at[d], recv_sem=orecv.at[my],
                device_id=(p,), device_id_type=pl.DeviceIdType.MESH,
            )
            ro.start()
            osends.append(ro)
        total = out_acc
        for p in _mesh_peers(my):
            ro = pltpu.make_async_remote_copy(
                src_ref=outr.at[p], dst_ref=outr.at[p],
                send_sem=osend.at[0], recv_sem=orecv.at[p],
                device_id=(p,), device_id_type=pl.DeviceIdType.MESH,
            )
            ro.wait_recv()
            total = total + outr[p]
        out_ref[0] = total

        for rdma in sends + osends:
            rdma.wait_send()

    return pl.pallas_call(
        body,
        out_shape=jax.ShapeDtypeStruct((1, SQ, DM), jnp.float32),
        in_specs=[
            pl.BlockSpec(memory_space=pltpu.VMEM),
            pl.BlockSpec(memory_space=pltpu.VMEM),
            pl.BlockSpec(memory_space=pltpu.VMEM),
        ],
        out_specs=pl.BlockSpec(memory_space=pltpu.VMEM),
        scratch_shapes=[
            pltpu.VMEM((N_DEV, HL, SQ, DH), jnp.float32),
            pltpu.VMEM((N_DEV, HL, 8, SQ), jnp.float32),
            pltpu.VMEM((N_DEV, SQ, DM), jnp.float32),
            pltpu.SemaphoreType.DMA((N_DEV - 1,)),
            pltpu.SemaphoreType.DMA((N_DEV,)),
            pltpu.SemaphoreType.DMA((N_DEV - 1,)),
            pltpu.SemaphoreType.DMA((N_DEV,)),
            pltpu.SemaphoreType.DMA((N_DEV - 1,)),
            pltpu.SemaphoreType.DMA((N_DEV,)),
        ],
        compiler_params=pltpu.CompilerParams(collective_id=1),
    )(num, den, Wo)


def kernel(x, Wq, K_ext, V_ext, Wo):
    q_all = _qproj_allgather(x, Wq)
    num, den = _local_attn(q_all, K_ext, V_ext)
    return _combine(num, den, Wo)


# device time: 128445 ns/iter; 1.7715x vs baseline; 1.7715x over previous
import jax
import jax.numpy as jnp
from jax import lax
from jax.experimental import pallas as pl
from jax.experimental.pallas import tpu as pltpu

N_DEV = 4
H = 32
HL = H // N_DEV
DH = 128
SQ = 256
SKV_LOC = 4096
DM = 1024
SCALE = 0.08838834764831843
QB = 64
NQB = SQ // QB


def _mesh_peers(my):
    return [(my + d) % N_DEV for d in range(1, N_DEV)]


def _entry_barrier(my):
    barrier = pltpu.get_barrier_semaphore()
    for p in _mesh_peers(my):
        pl.semaphore_signal(
            barrier, inc=1, device_id=(p,),
            device_id_type=pl.DeviceIdType.MESH,
        )
    pl.semaphore_wait(barrier, N_DEV - 1)


def _qproj_allgather(x, Wq):

    def body(x_ref, wq_ref, q_ref, send_sems, recv_sems):
        my = lax.axis_index("i")
        _entry_barrier(my)

        xq = x_ref[0].astype(jnp.bfloat16)
        wq = wq_ref[...].astype(jnp.bfloat16)
        q = lax.dot(xq, wq, preferred_element_type=jnp.float32)
        q_ref[my] = (q * SCALE).astype(jnp.bfloat16)

        sends = []
        for d, p in enumerate(_mesh_peers(my)):
            rdma = pltpu.make_async_remote_copy(
                src_ref=q_ref.at[my],
                dst_ref=q_ref.at[my],
                send_sem=send_sems.at[d],
                recv_sem=recv_sems.at[my],
                device_id=(p,),
                device_id_type=pl.DeviceIdType.MESH,
            )
            rdma.start()
            sends.append(rdma)
        for p in _mesh_peers(my):
            recv = pltpu.make_async_remote_copy(
                src_ref=q_ref.at[p],
                dst_ref=q_ref.at[p],
                send_sem=send_sems.at[0],
                recv_sem=recv_sems.at[p],
                device_id=(p,),
                device_id_type=pl.DeviceIdType.MESH,
            )
            recv.wait_recv()
        for rdma in sends:
            rdma.wait_send()

    return pl.pallas_call(
        body,
        out_shape=jax.ShapeDtypeStruct((N_DEV, SQ, DM), jnp.bfloat16),
        in_specs=[
            pl.BlockSpec(memory_space=pltpu.VMEM),
            pl.BlockSpec(memory_space=pltpu.VMEM),
        ],
        out_specs=pl.BlockSpec(memory_space=pltpu.VMEM),
        scratch_shapes=[
            pltpu.SemaphoreType.DMA((N_DEV - 1,)),
            pltpu.SemaphoreType.DMA((N_DEV,)),
        ],
        compiler_params=pltpu.CompilerParams(collective_id=0),
    )(x, Wq)


NKB = SKV_LOC // (4 * QB)


def _local_attn(q_all, K_ext, V_ext):

    def issue(hh, slot, k_hbm, v_hbm, kbuf, vbuf, ksems, vsems):
        for qb in range(NQB):
            pltpu.make_async_copy(
                k_hbm.at[:, qb, :, hh, :], kbuf.at[slot, qb], ksems.at[slot, qb]
            ).start()
            pltpu.make_async_copy(
                v_hbm.at[:, qb, :, hh, :], vbuf.at[slot, qb], vsems.at[slot, qb]
            ).start()

    def body(q_ref, k_hbm, v_hbm, num_ref, den_ref, kbuf, vbuf, ksems, vsems):
        h = pl.program_id(0)
        slot = lax.rem(h, 2)

        @pl.when(h == 0)
        def _():
            issue(h, slot, k_hbm, v_hbm, kbuf, vbuf, ksems, vsems)

        @pl.when(h + 1 < H)
        def _():
            issue(h + 1, lax.rem(h + 1, 2),
                  k_hbm, v_hbm, kbuf, vbuf, ksems, vsems)

        q = q_ref[0]
        for qb in range(NQB):
            pltpu.make_async_copy(
                k_hbm.at[:, qb, :, h, :], kbuf.at[slot, qb], ksems.at[slot, qb]
            ).wait()
            pltpu.make_async_copy(
                v_hbm.at[:, qb, :, h, :], vbuf.at[slot, qb], vsems.at[slot, qb]
            ).wait()
            qblk = q[qb * QB:(qb + 1) * QB, :]
            ksel = kbuf[slot, qb].reshape(-1, DH).astype(jnp.bfloat16)
            vsel = vbuf[slot, qb].reshape(-1, DH).astype(jnp.bfloat16)
            s = lax.dot_general(
                qblk, ksel, (((1,), (1,)), ((), ())),
                preferred_element_type=jnp.float32,
            )
            w = jnp.exp(s)
            den_ref[0, 0, qb * QB:(qb + 1) * QB] = jnp.sum(w, axis=1)
            num_ref[0, qb * QB:(qb + 1) * QB, :] = lax.dot(
                w.astype(jnp.bfloat16), vsel,
                preferred_element_type=jnp.float32,
            )

    return pl.pallas_call(
        body,
        grid=(H,),
        in_specs=[
            pl.BlockSpec((1, SQ, DH), lambda h: (h // HL, 0, h % HL)),
            pl.BlockSpec(memory_space=pl.ANY),
            pl.BlockSpec(memory_space=pl.ANY),
        ],
        out_specs=[
            pl.BlockSpec((1, SQ, DH), lambda h: (h, 0, 0)),
            pl.BlockSpec((1, 8, SQ), lambda h: (h, 0, 0)),
        ],
        out_shape=[
            jax.ShapeDtypeStruct((H, SQ, DH), jnp.float32),
            jax.ShapeDtypeStruct((H, 8, SQ), jnp.float32),
        ],
        scratch_shapes=[
            pltpu.VMEM((2, NQB, NKB, QB, DH), jnp.float32),
            pltpu.VMEM((2, NQB, NKB, QB, DH), jnp.float32),
            pltpu.SemaphoreType.DMA((2, NQB)),
            pltpu.SemaphoreType.DMA((2, NQB)),
        ],
    )(
        q_all,
        K_ext.reshape(NKB, NQB, QB, H, DH),
        V_ext.reshape(NKB, NQB, QB, H, DH),
    )


def _combine(num, den, Wo):

    def body(num_ref, den_ref, wo_ref, out_ref,
             numr, denr, outr,
             nsend, nrecv, dsend, drecv, osend, orecv):
        my = lax.axis_index("i")
        _entry_barrier(my)

        sends = []
        for d, p in enumerate(_mesh_peers(my)):
            rn = pltpu.make_async_remote_copy(
                src_ref=num_ref.at[pl.ds(p * HL, HL)],
                dst_ref=numr.at[my],
                send_sem=nsend.at[d], recv_sem=nrecv.at[my],
                device_id=(p,), device_id_type=pl.DeviceIdType.MESH,
            )
            rn.start()
            rd = pltpu.make_async_remote_copy(
                src_ref=den_ref.at[pl.ds(p * HL, HL)],
                dst_ref=denr.at[my],
                send_sem=dsend.at[d], recv_sem=drecv.at[my],
                device_id=(p,), device_id_type=pl.DeviceIdType.MESH,
            )
            rd.start()
            sends += [rn, rd]

        acc_n = num_ref[pl.ds(my * HL, HL)]
        acc_d = den_ref[pl.ds(my * HL, HL), 0, :]
        for p in _mesh_peers(my):
            rn = pltpu.make_async_remote_copy(
                src_ref=num_ref.at[pl.ds(0, HL)], dst_ref=numr.at[p],
                send_sem=nsend.at[0], recv_sem=nrecv.at[p],
                device_id=(p,), device_id_type=pl.DeviceIdType.MESH,
            )
            rn.wait_recv()
            rd = pltpu.make_async_remote_copy(
                src_ref=den_ref.at[pl.ds(0, HL)], dst_ref=denr.at[p],
                send_sem=dsend.at[0], recv_sem=drecv.at[p],
                device_id=(p,), device_id_type=pl.DeviceIdType.MESH,
            )
            rd.wait_recv()
            acc_n = acc_n + numr[p]
            acc_d = acc_d + denr[p, :, 0, :]

        ctx = (acc_n / acc_d[:, :, None]).astype(jnp.bfloat16)
        wo = wo_ref[...].astype(jnp.bfloat16)
        out_acc = jnp.zeros((SQ, DM), jnp.float32)
        for h in range(HL):
            out_acc = out_acc + lax.dot(
                ctx[h], wo[h * DH:(h + 1) * DH, :],
                preferred_element_type=jnp.float32,
            )
        outr[my] = out_acc

        osends = []
        for d, p in enumerate(_mesh_peers(my)):
            ro = pltpu.make_async_remote_copy(
                src_ref=outr.at[my], dst_ref=outr.at[my],
                send_sem=osend.at[d], recv_sem=orecv.at[my],
                device_id=(p,), device_id_type=pl.DeviceIdType.MESH,
            )
            ro.start()
            osends.append(ro)
        total = out_acc
        for p in _mesh_peers(my):
            ro = pltpu.make_async_remote_copy(
                src_ref=outr.at[p], dst_ref=outr.at[p],
                send_sem=osend.at[0], recv_sem=orecv.at[p],
                device_id=(p,), device_id_type=pl.DeviceIdType.MESH,
            )
            ro.wait_recv()
            total = total + outr[p]
        out_ref[0] = total

        for rdma in sends + osends:
            rdma.wait_send()

    return pl.pallas_call(
        body,
        out_shape=jax.ShapeDtypeStruct((1, SQ, DM), jnp.float32),
        in_specs=[
            pl.BlockSpec(memory_space=pltpu.VMEM),
            pl.BlockSpec(memory_space=pltpu.VMEM),
            pl.BlockSpec(memory_space=pltpu.VMEM),
        ],
        out_specs=pl.BlockSpec(memory_space=pltpu.VMEM),
        scratch_shapes=[
            pltpu.VMEM((N_DEV, HL, SQ, DH), jnp.float32),
            pltpu.VMEM((N_DEV, HL, 8, SQ), jnp.float32),
            pltpu.VMEM((N_DEV, SQ, DM), jnp.float32),
            pltpu.SemaphoreType.DMA((N_DEV - 1,)),
            pltpu.SemaphoreType.DMA((N_DEV,)),
            pltpu.SemaphoreType.DMA((N_DEV - 1,)),
            pltpu.SemaphoreType.DMA((N_DEV,)),
            pltpu.SemaphoreType.DMA((N_DEV - 1,)),
            pltpu.SemaphoreType.DMA((N_DEV,)),
        ],
        compiler_params=pltpu.CompilerParams(collective_id=1),
    )(num, den, Wo)


def kernel(x, Wq, K_ext, V_ext, Wo):
    q_all = _qproj_allgather(x, Wq)
    num, den = _local_attn(q_all, K_ext, V_ext)
    return _combine(num, den, Wo)
